# Initial kernel scaffold; baseline (speedup 1.0000x reference)
import jax
import jax.numpy as jnp
from jax.experimental import pallas as pl


def kernel(edge_index, edge_type, dynamic_emb, emb_rel, W_ih, W_hh, b_ih, b_hh, rgcn_W, time_gate_weight, time_gate_bias):
    raise NotImplementedError("write your pallas kernel here")


# speedup vs baseline: 2.9986x; 2.9986x over previous
import functools

import jax
import jax.numpy as jnp
from jax import lax
from jax.experimental import pallas as pl
from jax.experimental.pallas import tpu as pltpu
from jax.experimental.pallas import tpu_sc as plsc

NUM_ENTS = 10000
H = 128
N_EDGES = 320000
NR2 = 400

NC = 2
NS = 16
NW = NC * NS
CH = 80
EPW = N_EDGES // NW
NCHUNK = EPW // CH

AGGN = 10240
AGG_PT = AGGN // NS
RELN = 512
REL_PT = RELN // NS
CNTN = 10496
CNT_PT = CNTN // NS
G = 5

_RRELU_SLOPE = (1.0 / 8.0 + 1.0 / 3.0) / 2.0

_MESH = plsc.VectorSubcoreMesh(
    core_axis_name="c", subcore_axis_name="s", num_cores=NC, num_subcores=NS)


def _zero_vmem_f32(ref, nrows, ncols):
    z = jnp.zeros((16,), jnp.float32)

    @pl.loop(0, nrows * (ncols // 16))
    def _(t):
        i = t // (ncols // 16)
        k = (t % (ncols // 16)) * 16
        ref[i, pl.ds(k, 16)] = z


@functools.partial(
    pl.kernel,
    out_type=(
        jax.ShapeDtypeStruct((NC, AGGN, H), jnp.float32),
        jax.ShapeDtypeStruct((NC, RELN, H), jnp.float32),
        jax.ShapeDtypeStruct((NC, CNTN, 16), jnp.float32),
    ),
    mesh=_MESH,
    compiler_params=pltpu.CompilerParams(use_tc_tiling_on_sc=False),
    scratch_types=[
        pltpu.VMEM((G, CH), jnp.int32),
        pltpu.VMEM((G, CH), jnp.int32),
        pltpu.VMEM((G, CH), jnp.int32),
        pltpu.VMEM((G, CH), jnp.int32),
        pltpu.VMEM((CH, H), jnp.float32),
        pltpu.VMEM((CH, 16), jnp.float32),
        pltpu.VMEM_SHARED((AGGN, H), jnp.float32),
        pltpu.VMEM_SHARED((RELN, H), jnp.float32),
        pltpu.VMEM_SHARED((CNTN, 16), jnp.float32),
        pltpu.SemaphoreType.DMA,
    ],
)
def _stage_a(h_hbm, src_hbm, dst_hbm, et_hbm, etoff_hbm,
             agg_out, rel_out, cnt_out,
             idx_src, idx_dst, idx_et, idx_etoff, rows, ones_v,
             agg_sh, rel_sh, cnt_sh, sem):
    c = lax.axis_index("c")
    s = lax.axis_index("s")
    wid = s * NC + c

    _zero_vmem_f32(rows, CH, H)
    for j in range(AGG_PT // CH):
        pltpu.sync_copy(rows, agg_sh.at[pl.ds(s * AGG_PT + j * CH, CH), :])
    pltpu.sync_copy(rows.at[pl.ds(0, REL_PT), :],
                    rel_sh.at[pl.ds(s * REL_PT, REL_PT), :])
    for j in range(CNT_PT // CH):
        pltpu.sync_copy(rows.at[:, pl.ds(0, 16)],
                        cnt_sh.at[pl.ds(s * CNT_PT + j * CH, CH), :])
    pltpu.sync_copy(rows.at[pl.ds(0, CNT_PT % CH), pl.ds(0, 16)],
                    cnt_sh.at[pl.ds(s * CNT_PT + (CNT_PT // CH) * CH,
                                    CNT_PT % CH), :])

    one_row = jnp.where(lax.iota(jnp.int32, 16) == 0, 1.0, 0.0)

    @pl.loop(0, CH)
    def _(i):
        ones_v[i, :] = one_row

    plsc.subcore_barrier()

    @pl.loop(0, NCHUNK // G)
    def _(o):
        pltpu.sync_copy(src_hbm.at[wid, pl.ds(o * G, G), :], idx_src)
        pltpu.sync_copy(dst_hbm.at[wid, pl.ds(o * G, G), :], idx_dst)
        pltpu.sync_copy(et_hbm.at[wid, pl.ds(o * G, G), :], idx_et)
        pltpu.sync_copy(etoff_hbm.at[wid, pl.ds(o * G, G), :], idx_etoff)
        for g in range(G):
            pltpu.async_copy(h_hbm.at[idx_src.at[g]], rows, sem).wait()
            pltpu.sync_copy(rows, agg_sh.at[idx_dst.at[g]], add=True)
            pltpu.sync_copy(rows, rel_sh.at[idx_et.at[g]], add=True)
            pltpu.sync_copy(ones_v, cnt_sh.at[idx_dst.at[g]], add=True)
            pltpu.sync_copy(ones_v, cnt_sh.at[idx_etoff.at[g]], add=True)

    plsc.subcore_barrier()
    pltpu.sync_copy(agg_sh.at[pl.ds(s * AGG_PT, AGG_PT), :],
                    agg_out.at[c, pl.ds(s * AGG_PT, AGG_PT), :])
    pltpu.sync_copy(rel_sh.at[pl.ds(s * REL_PT, REL_PT), :],
                    rel_out.at[c, pl.ds(s * REL_PT, REL_PT), :])
    pltpu.sync_copy(cnt_sh.at[pl.ds(s * CNT_PT, CNT_PT), :],
                    cnt_out.at[c, pl.ds(s * CNT_PT, CNT_PT), :])


@functools.partial(
    pl.kernel,
    out_type=jax.ShapeDtypeStruct((NC, AGGN, H), jnp.float32),
    mesh=_MESH,
    compiler_params=pltpu.CompilerParams(use_tc_tiling_on_sc=False),
    scratch_types=[
        pltpu.VMEM((G, CH), jnp.int32),
        pltpu.VMEM((G, CH), jnp.int32),
        pltpu.VMEM((CH, H), jnp.float32),
        pltpu.VMEM_SHARED((AGGN, H), jnp.float32),
        pltpu.SemaphoreType.DMA,
    ],
)
def _stage_c(rel_hbm, dst_hbm, et_hbm, agg1_hbm, agg_out,
             idx_dst, idx_et, rows, agg_sh, sem):
    c = lax.axis_index("c")
    s = lax.axis_index("s")
    wid = s * NC + c

    pltpu.sync_copy(agg1_hbm.at[c, pl.ds(s * AGG_PT, AGG_PT), :],
                    agg_sh.at[pl.ds(s * AGG_PT, AGG_PT), :])
    plsc.subcore_barrier()

    @pl.loop(0, NCHUNK // G)
    def _(o):
        pltpu.sync_copy(dst_hbm.at[wid, pl.ds(o * G, G), :], idx_dst)
        pltpu.sync_copy(et_hbm.at[wid, pl.ds(o * G, G), :], idx_et)
        for g in range(G):
            pltpu.async_copy(rel_hbm.at[idx_et.at[g]], rows, sem).wait()
            pltpu.sync_copy(rows, agg_sh.at[idx_dst.at[g]], add=True)

    plsc.subcore_barrier()
    pltpu.sync_copy(agg_sh.at[pl.ds(s * AGG_PT, AGG_PT), :],
                    agg_out.at[c, pl.ds(s * AGG_PT, AGG_PT), :])


def _gru_body(emb_rel_ref, relsum_ref, relcnt_ref, w_ih_ref, w_hh_ref,
              b_ih_ref, b_hh_ref, out_ref):
    emb_rel = emb_rel_ref[...]
    rs = relsum_ref[0] + relsum_ref[1]
    cnt = relcnt_ref[...]
    rc = cnt[0, :, 0] + cnt[1, :, 0]
    x_input = rs / jnp.maximum(rc, 1.0)[:, None]
    x_cat = jnp.concatenate([emb_rel, x_input], axis=1)
    gi = lax.dot_general(x_cat, w_ih_ref[...], (((1,), (1,)), ((), ())),
                         preferred_element_type=jnp.float32) + b_ih_ref[...]
    gh = lax.dot_general(emb_rel, w_hh_ref[...], (((1,), (1,)), ((), ())),
                         preferred_element_type=jnp.float32) + b_hh_ref[...]
    r = jax.nn.sigmoid(gi[:, :H] + gh[:, :H])
    z = jax.nn.sigmoid(gi[:, H:2 * H] + gh[:, H:2 * H])
    n = jnp.tanh(gi[:, 2 * H:] + r * gh[:, 2 * H:])
    out_ref[...] = (1.0 - z) * n + z * emb_rel


_FINAL_R = 1000


def _final_body(agg2_ref, cnt_ref, h_ref, w_ref, tgw_ref, tgb_ref, out_ref):
    cnt = cnt_ref[...]
    deg = cnt[0, :, 0] + cnt[1, :, 0]
    agg = (agg2_ref[0] + agg2_ref[1]) / jnp.maximum(deg, 1.0)[:, None]
    hn = lax.dot_general(agg, w_ref[...], (((1,), (0,)), ((), ())),
                         preferred_element_type=jnp.float32)
    hn = jnp.where(hn >= 0, hn, _RRELU_SLOPE * hn)
    gate = jax.nn.sigmoid(
        lax.dot_general(hn, tgw_ref[...], (((1,), (0,)), ((), ())),
                        preferred_element_type=jnp.float32) + tgb_ref[...])
    out_ref[...] = gate * hn + (1.0 - gate) * h_ref[...]


def kernel(edge_index, edge_type, dynamic_emb, emb_rel, W_ih, W_hh, b_ih, b_hh,
           rgcn_W, time_gate_weight, time_gate_bias):
    src = edge_index[0].reshape(NW, NCHUNK, CH)
    dst = edge_index[1].reshape(NW, NCHUNK, CH)
    et = edge_type.reshape(NW, NCHUNK, CH)
    etoff = (edge_type + NUM_ENTS).reshape(NW, NCHUNK, CH)

    agg1, relsum, cnt = _stage_a(dynamic_emb, src, dst, et, etoff)

    rel_evolved = pl.pallas_call(
        _gru_body,
        out_shape=jax.ShapeDtypeStruct((NR2, H), jnp.float32),
    )(emb_rel, relsum[:, :NR2, :], cnt[:, NUM_ENTS:NUM_ENTS + NR2, :],
      W_ih, W_hh, b_ih.reshape(1, 3 * H), b_hh.reshape(1, 3 * H))

    agg2 = _stage_c(rel_evolved, dst, et, agg1)

    out = pl.pallas_call(
        _final_body,
        grid=(NUM_ENTS // _FINAL_R,),
        in_specs=[
            pl.BlockSpec((NC, _FINAL_R, H), lambda i: (0, i, 0)),
            pl.BlockSpec((NC, _FINAL_R, 16), lambda i: (0, i, 0)),
            pl.BlockSpec((_FINAL_R, H), lambda i: (i, 0)),
            pl.BlockSpec((H, H), lambda i: (0, 0)),
            pl.BlockSpec((H, H), lambda i: (0, 0)),
            pl.BlockSpec((1, H), lambda i: (0, 0)),
        ],
        out_specs=pl.BlockSpec((_FINAL_R, H), lambda i: (i, 0)),
        out_shape=jax.ShapeDtypeStruct((NUM_ENTS, H), jnp.float32),
    )(agg2, cnt, dynamic_emb, rgcn_W, time_gate_weight,
      time_gate_bias.reshape(1, H))
    return out


# speedup vs baseline: 3.0202x; 1.0072x over previous
import functools

import jax
import jax.numpy as jnp
from jax import lax
from jax.experimental import pallas as pl
from jax.experimental.pallas import tpu as pltpu
from jax.experimental.pallas import tpu_sc as plsc

NUM_ENTS = 10000
H = 128
N_EDGES = 320000
NR2 = 400

NC = 2
NS = 16
NW = NC * NS
CH = 80
EPW = N_EDGES // NW
NCHUNK = EPW // CH

AGGN = 10240
AGG_PT = AGGN // NS
RELN = 512
REL_PT = RELN // NS
CNTN = 10496
CNT_PT = CNTN // NS

_RRELU_SLOPE = (1.0 / 8.0 + 1.0 / 3.0) / 2.0

_MESH = plsc.VectorSubcoreMesh(
    core_axis_name="c", subcore_axis_name="s", num_cores=NC, num_subcores=NS)


def _zero_vmem_f32(ref, nrows, ncols):
    z = jnp.zeros((16,), jnp.float32)

    @pl.loop(0, nrows * (ncols // 16))
    def _(t):
        i = t // (ncols // 16)
        k = (t % (ncols // 16)) * 16
        ref[i, pl.ds(k, 16)] = z


@functools.partial(
    pl.kernel,
    out_type=(
        jax.ShapeDtypeStruct((NC, AGGN, H), jnp.float32),
        jax.ShapeDtypeStruct((NC, RELN, H), jnp.float32),
        jax.ShapeDtypeStruct((NC, CNTN, 16), jnp.float32),
    ),
    mesh=_MESH,
    compiler_params=pltpu.CompilerParams(use_tc_tiling_on_sc=False),
    scratch_types=[
        pltpu.VMEM((2, CH), jnp.int32),
        pltpu.VMEM((2, CH), jnp.int32),
        pltpu.VMEM((2, CH), jnp.int32),
        pltpu.VMEM((2, CH), jnp.int32),
        pltpu.VMEM((CH, H), jnp.float32),
        pltpu.VMEM((CH, H), jnp.float32),
        pltpu.VMEM((CH, 16), jnp.float32),
        pltpu.VMEM_SHARED((AGGN, H), jnp.float32),
        pltpu.VMEM_SHARED((RELN, H), jnp.float32),
        pltpu.VMEM_SHARED((CNTN, 16), jnp.float32),
        pltpu.SemaphoreType.DMA,
        pltpu.SemaphoreType.DMA,
        pltpu.SemaphoreType.DMA,
    ],
)
def _stage_a(h_hbm, src_hbm, dst_hbm, et_hbm, etoff_hbm,
             agg_out, rel_out, cnt_out,
             idx_src, idx_dst, idx_et, idx_etoff, rows, rows1, ones_v,
             agg_sh, rel_sh, cnt_sh, sem_g0, sem_g1, sem_s):
    c = lax.axis_index("c")
    s = lax.axis_index("s")
    wid = s * NC + c

    _zero_vmem_f32(rows, CH, H)
    for j in range(AGG_PT // CH):
        pltpu.sync_copy(rows, agg_sh.at[pl.ds(s * AGG_PT + j * CH, CH), :])
    pltpu.sync_copy(rows.at[pl.ds(0, REL_PT), :],
                    rel_sh.at[pl.ds(s * REL_PT, REL_PT), :])
    for j in range(CNT_PT // CH):
        pltpu.sync_copy(rows.at[:, pl.ds(0, 16)],
                        cnt_sh.at[pl.ds(s * CNT_PT + j * CH, CH), :])
    pltpu.sync_copy(rows.at[pl.ds(0, CNT_PT % CH), pl.ds(0, 16)],
                    cnt_sh.at[pl.ds(s * CNT_PT + (CNT_PT // CH) * CH,
                                    CNT_PT % CH), :])

    one_row = jnp.where(lax.iota(jnp.int32, 16) == 0, 1.0, 0.0)

    @pl.loop(0, CH)
    def _(i):
        ones_v[i, :] = one_row

    plsc.subcore_barrier()

    @pl.loop(0, NCHUNK // 2)
    def _(o):
        j0 = o * 2
        pltpu.sync_copy(src_hbm.at[wid, pl.ds(j0, 2), :], idx_src)
        pltpu.sync_copy(dst_hbm.at[wid, pl.ds(j0, 2), :], idx_dst)
        pltpu.sync_copy(et_hbm.at[wid, pl.ds(j0, 2), :], idx_et)
        pltpu.sync_copy(etoff_hbm.at[wid, pl.ds(j0, 2), :], idx_etoff)
        g0 = pltpu.async_copy(h_hbm.at[idx_src.at[0]], rows, sem_g0)
        g1 = pltpu.async_copy(h_hbm.at[idx_src.at[1]], rows1, sem_g1)
        g0.wait()
        pend = [
            pltpu.async_copy(rows, agg_sh.at[idx_dst.at[0]], sem_s, add=True),
            pltpu.async_copy(rows, rel_sh.at[idx_et.at[0]], sem_s, add=True),
            pltpu.async_copy(ones_v, cnt_sh.at[idx_dst.at[0]], sem_s, add=True),
            pltpu.async_copy(ones_v, cnt_sh.at[idx_etoff.at[0]], sem_s,
                             add=True),
        ]
        g1.wait()
        pend += [
            pltpu.async_copy(rows1, agg_sh.at[idx_dst.at[1]], sem_s, add=True),
            pltpu.async_copy(rows1, rel_sh.at[idx_et.at[1]], sem_s, add=True),
            pltpu.async_copy(ones_v, cnt_sh.at[idx_dst.at[1]], sem_s, add=True),
            pltpu.async_copy(ones_v, cnt_sh.at[idx_etoff.at[1]], sem_s,
                             add=True),
        ]
        for p in pend:
            p.wait()

    jt = NCHUNK - 1
    pltpu.sync_copy(src_hbm.at[wid, pl.ds(jt, 1), :], idx_src.at[pl.ds(0, 1)])
    pltpu.sync_copy(dst_hbm.at[wid, pl.ds(jt, 1), :], idx_dst.at[pl.ds(0, 1)])
    pltpu.sync_copy(et_hbm.at[wid, pl.ds(jt, 1), :], idx_et.at[pl.ds(0, 1)])
    pltpu.sync_copy(etoff_hbm.at[wid, pl.ds(jt, 1), :],
                    idx_etoff.at[pl.ds(0, 1)])
    pltpu.async_copy(h_hbm.at[idx_src.at[0]], rows, sem_g0).wait()
    pltpu.sync_copy(rows, agg_sh.at[idx_dst.at[0]], add=True)
    pltpu.sync_copy(rows, rel_sh.at[idx_et.at[0]], add=True)
    pltpu.sync_copy(ones_v, cnt_sh.at[idx_dst.at[0]], add=True)
    pltpu.sync_copy(ones_v, cnt_sh.at[idx_etoff.at[0]], add=True)

    plsc.subcore_barrier()
    pltpu.sync_copy(agg_sh.at[pl.ds(s * AGG_PT, AGG_PT), :],
                    agg_out.at[c, pl.ds(s * AGG_PT, AGG_PT), :])
    pltpu.sync_copy(rel_sh.at[pl.ds(s * REL_PT, REL_PT), :],
                    rel_out.at[c, pl.ds(s * REL_PT, REL_PT), :])
    pltpu.sync_copy(cnt_sh.at[pl.ds(s * CNT_PT, CNT_PT), :],
                    cnt_out.at[c, pl.ds(s * CNT_PT, CNT_PT), :])


@functools.partial(
    pl.kernel,
    out_type=jax.ShapeDtypeStruct((NC, AGGN, H), jnp.float32),
    mesh=_MESH,
    compiler_params=pltpu.CompilerParams(use_tc_tiling_on_sc=False),
    scratch_types=[
        pltpu.VMEM((2, CH), jnp.int32),
        pltpu.VMEM((2, CH), jnp.int32),
        pltpu.VMEM((CH, H), jnp.float32),
        pltpu.VMEM((CH, H), jnp.float32),
        pltpu.VMEM_SHARED((AGGN, H), jnp.float32),
        pltpu.SemaphoreType.DMA,
        pltpu.SemaphoreType.DMA,
        pltpu.SemaphoreType.DMA,
    ],
)
def _stage_c(rel_hbm, dst_hbm, et_hbm, agg1_hbm, agg_out,
             idx_dst, idx_et, rows, rows1, agg_sh, sem_g0, sem_g1, sem_s):
    c = lax.axis_index("c")
    s = lax.axis_index("s")
    wid = s * NC + c

    pltpu.sync_copy(agg1_hbm.at[c, pl.ds(s * AGG_PT, AGG_PT), :],
                    agg_sh.at[pl.ds(s * AGG_PT, AGG_PT), :])
    plsc.subcore_barrier()

    @pl.loop(0, NCHUNK // 2)
    def _(o):
        j0 = o * 2
        pltpu.sync_copy(dst_hbm.at[wid, pl.ds(j0, 2), :], idx_dst)
        pltpu.sync_copy(et_hbm.at[wid, pl.ds(j0, 2), :], idx_et)
        g0 = pltpu.async_copy(rel_hbm.at[idx_et.at[0]], rows, sem_g0)
        g1 = pltpu.async_copy(rel_hbm.at[idx_et.at[1]], rows1, sem_g1)
        g0.wait()
        s0 = pltpu.async_copy(rows, agg_sh.at[idx_dst.at[0]], sem_s, add=True)
        g1.wait()
        s1 = pltpu.async_copy(rows1, agg_sh.at[idx_dst.at[1]], sem_s, add=True)
        s0.wait()
        s1.wait()

    jt = NCHUNK - 1
    pltpu.sync_copy(dst_hbm.at[wid, pl.ds(jt, 1), :], idx_dst.at[pl.ds(0, 1)])
    pltpu.sync_copy(et_hbm.at[wid, pl.ds(jt, 1), :], idx_et.at[pl.ds(0, 1)])
    pltpu.async_copy(rel_hbm.at[idx_et.at[0]], rows, sem_g0).wait()
    pltpu.sync_copy(rows, agg_sh.at[idx_dst.at[0]], add=True)

    plsc.subcore_barrier()
    pltpu.sync_copy(agg_sh.at[pl.ds(s * AGG_PT, AGG_PT), :],
                    agg_out.at[c, pl.ds(s * AGG_PT, AGG_PT), :])


def _gru_body(emb_rel_ref, relsum_ref, relcnt_ref, w_ih_ref, w_hh_ref,
              b_ih_ref, b_hh_ref, out_ref):
    emb_rel = emb_rel_ref[...]
    rs = relsum_ref[0] + relsum_ref[1]
    cnt = relcnt_ref[...]
    rc = cnt[0, :, 0] + cnt[1, :, 0]
    x_input = rs / jnp.maximum(rc, 1.0)[:, None]
    x_cat = jnp.concatenate([emb_rel, x_input], axis=1)
    gi = lax.dot_general(x_cat, w_ih_ref[...], (((1,), (1,)), ((), ())),
                         preferred_element_type=jnp.float32) + b_ih_ref[...]
    gh = lax.dot_general(emb_rel, w_hh_ref[...], (((1,), (1,)), ((), ())),
                         preferred_element_type=jnp.float32) + b_hh_ref[...]
    r = jax.nn.sigmoid(gi[:, :H] + gh[:, :H])
    z = jax.nn.sigmoid(gi[:, H:2 * H] + gh[:, H:2 * H])
    n = jnp.tanh(gi[:, 2 * H:] + r * gh[:, 2 * H:])
    out_ref[...] = (1.0 - z) * n + z * emb_rel


_FINAL_R = 1000


def _final_body(agg2_ref, cnt_ref, h_ref, w_ref, tgw_ref, tgb_ref, out_ref):
    cnt = cnt_ref[...]
    deg = cnt[0, :, 0] + cnt[1, :, 0]
    agg = (agg2_ref[0] + agg2_ref[1]) / jnp.maximum(deg, 1.0)[:, None]
    hn = lax.dot_general(agg, w_ref[...], (((1,), (0,)), ((), ())),
                         preferred_element_type=jnp.float32)
    hn = jnp.where(hn >= 0, hn, _RRELU_SLOPE * hn)
    gate = jax.nn.sigmoid(
        lax.dot_general(hn, tgw_ref[...], (((1,), (0,)), ((), ())),
                        preferred_element_type=jnp.float32) + tgb_ref[...])
    out_ref[...] = gate * hn + (1.0 - gate) * h_ref[...]


def kernel(edge_index, edge_type, dynamic_emb, emb_rel, W_ih, W_hh, b_ih, b_hh,
           rgcn_W, time_gate_weight, time_gate_bias):
    src = edge_index[0].reshape(NW, NCHUNK, CH)
    dst = edge_index[1].reshape(NW, NCHUNK, CH)
    et = edge_type.reshape(NW, NCHUNK, CH)
    etoff = (edge_type + NUM_ENTS).reshape(NW, NCHUNK, CH)

    agg1, relsum, cnt = _stage_a(dynamic_emb, src, dst, et, etoff)

    rel_evolved = pl.pallas_call(
        _gru_body,
        out_shape=jax.ShapeDtypeStruct((NR2, H), jnp.float32),
    )(emb_rel, relsum[:, :NR2, :], cnt[:, NUM_ENTS:NUM_ENTS + NR2, :],
      W_ih, W_hh, b_ih.reshape(1, 3 * H), b_hh.reshape(1, 3 * H))

    agg2 = _stage_c(rel_evolved, dst, et, agg1)

    out = pl.pallas_call(
        _final_body,
        grid=(NUM_ENTS // _FINAL_R,),
        in_specs=[
            pl.BlockSpec((NC, _FINAL_R, H), lambda i: (0, i, 0)),
            pl.BlockSpec((NC, _FINAL_R, 16), lambda i: (0, i, 0)),
            pl.BlockSpec((_FINAL_R, H), lambda i: (i, 0)),
            pl.BlockSpec((H, H), lambda i: (0, 0)),
            pl.BlockSpec((H, H), lambda i: (0, 0)),
            pl.BlockSpec((1, H), lambda i: (0, 0)),
        ],
        out_specs=pl.BlockSpec((_FINAL_R, H), lambda i: (i, 0)),
        out_shape=jax.ShapeDtypeStruct((NUM_ENTS, H), jnp.float32),
    )(agg2, cnt, dynamic_emb, rgcn_W, time_gate_weight,
      time_gate_bias.reshape(1, H))
    return out


# speedup vs baseline: 3.1338x; 1.0376x over previous
import functools

import jax
import jax.numpy as jnp
from jax import lax
from jax.experimental import pallas as pl
from jax.experimental.pallas import tpu as pltpu
from jax.experimental.pallas import tpu_sc as plsc

NUM_ENTS = 10000
H = 128
N_EDGES = 320000
NR2 = 400

NC = 2
NS = 16
NW = NC * NS
CH = 80
EPW = N_EDGES // NW
NCHUNK = EPW // CH

AGGN = 10240
AGG_PT = AGGN // NS
RELN = 512
REL_PT = RELN // NS
CNTN = 10400

_RRELU_SLOPE = (1.0 / 8.0 + 1.0 / 3.0) / 2.0

_MESH = plsc.VectorSubcoreMesh(
    core_axis_name="c", subcore_axis_name="s", num_cores=NC, num_subcores=NS)


def _zero_vmem_f32(ref, nrows, ncols):
    z = jnp.zeros((16,), jnp.float32)

    @pl.loop(0, nrows * (ncols // 16))
    def _(t):
        i = t // (ncols // 16)
        k = (t % (ncols // 16)) * 16
        ref[i, pl.ds(k, 16)] = z


@functools.partial(
    pl.kernel,
    out_type=(
        jax.ShapeDtypeStruct((NC, AGGN, H), jnp.float32),
        jax.ShapeDtypeStruct((NC, RELN, H), jnp.float32),
        jax.ShapeDtypeStruct((NC, NS, CNTN), jnp.float32),
    ),
    mesh=_MESH,
    compiler_params=pltpu.CompilerParams(use_tc_tiling_on_sc=False, needs_layout_passes=False),
    scratch_types=[
        pltpu.VMEM((2, CH), jnp.int32),
        pltpu.VMEM((2, CH), jnp.int32),
        pltpu.VMEM((2, CH), jnp.int32),
        pltpu.VMEM((CH, H), jnp.float32),
        pltpu.VMEM((CH, H), jnp.float32),
        pltpu.VMEM((CNTN,), jnp.float32),
        pltpu.VMEM_SHARED((AGGN, H), jnp.float32),
        pltpu.VMEM_SHARED((RELN, H), jnp.float32),
        pltpu.SemaphoreType.DMA,
        pltpu.SemaphoreType.DMA,
        pltpu.SemaphoreType.DMA,
    ],
)
def _stage_a(h_hbm, src_hbm, dst_hbm, et_hbm,
             agg_out, rel_out, cnt_out,
             idx_src, idx_dst, idx_et, rows, rows1, cnt_v,
             agg_sh, rel_sh, sem_g0, sem_g1, sem_s):
    c = lax.axis_index("c")
    s = lax.axis_index("s")
    wid = s * NC + c

    _zero_vmem_f32(rows, CH, H)
    for j in range(AGG_PT // CH):
        pltpu.sync_copy(rows, agg_sh.at[pl.ds(s * AGG_PT + j * CH, CH), :])
    pltpu.sync_copy(rows.at[pl.ds(0, REL_PT), :],
                    rel_sh.at[pl.ds(s * REL_PT, REL_PT), :])

    z = jnp.zeros((16,), jnp.float32)

    @pl.loop(0, CNTN // 16)
    def _(t):
        cnt_v[pl.ds(t * 16, 16)] = z

    plsc.subcore_barrier()

    ones16 = jnp.full((16,), 1.0, jnp.float32)

    def _count(buf_row):
        for v in range(CH // 16):
            d = idx_dst[buf_row, pl.ds(v * 16, 16)]
            plsc.addupdate_scatter(cnt_v, [d], ones16)
            e = idx_et[buf_row, pl.ds(v * 16, 16)] + NUM_ENTS
            plsc.addupdate_scatter(cnt_v, [e], ones16)

    @pl.loop(0, NCHUNK // 2)
    def _(o):
        j0 = o * 2
        pltpu.sync_copy(src_hbm.at[wid, pl.ds(j0, 2), :], idx_src)
        pltpu.sync_copy(dst_hbm.at[wid, pl.ds(j0, 2), :], idx_dst)
        pltpu.sync_copy(et_hbm.at[wid, pl.ds(j0, 2), :], idx_et)
        g0 = pltpu.async_copy(h_hbm.at[idx_src.at[0]], rows, sem_g0)
        g1 = pltpu.async_copy(h_hbm.at[idx_src.at[1]], rows1, sem_g1)
        g0.wait()
        pend = [
            pltpu.async_copy(rows, agg_sh.at[idx_dst.at[0]], sem_s, add=True),
            pltpu.async_copy(rows, rel_sh.at[idx_et.at[0]], sem_s, add=True),
        ]
        g1.wait()
        pend += [
            pltpu.async_copy(rows1, agg_sh.at[idx_dst.at[1]], sem_s, add=True),
            pltpu.async_copy(rows1, rel_sh.at[idx_et.at[1]], sem_s, add=True),
        ]
        _count(0)
        _count(1)
        for p in pend:
            p.wait()

    jt = NCHUNK - 1
    pltpu.sync_copy(src_hbm.at[wid, pl.ds(jt, 1), :], idx_src.at[pl.ds(0, 1)])
    pltpu.sync_copy(dst_hbm.at[wid, pl.ds(jt, 1), :], idx_dst.at[pl.ds(0, 1)])
    pltpu.sync_copy(et_hbm.at[wid, pl.ds(jt, 1), :], idx_et.at[pl.ds(0, 1)])
    pltpu.async_copy(h_hbm.at[idx_src.at[0]], rows, sem_g0).wait()
    pltpu.sync_copy(rows, agg_sh.at[idx_dst.at[0]], add=True)
    pltpu.sync_copy(rows, rel_sh.at[idx_et.at[0]], add=True)
    _count(0)

    plsc.subcore_barrier()
    pltpu.sync_copy(agg_sh.at[pl.ds(s * AGG_PT, AGG_PT), :],
                    agg_out.at[c, pl.ds(s * AGG_PT, AGG_PT), :])
    pltpu.sync_copy(rel_sh.at[pl.ds(s * REL_PT, REL_PT), :],
                    rel_out.at[c, pl.ds(s * REL_PT, REL_PT), :])
    pltpu.sync_copy(cnt_v, cnt_out.at[c, s, :])


@functools.partial(
    pl.kernel,
    out_type=jax.ShapeDtypeStruct((NC, AGGN, H), jnp.float32),
    mesh=_MESH,
    compiler_params=pltpu.CompilerParams(use_tc_tiling_on_sc=False, needs_layout_passes=False),
    scratch_types=[
        pltpu.VMEM((2, CH), jnp.int32),
        pltpu.VMEM((2, CH), jnp.int32),
        pltpu.VMEM((CH, H), jnp.float32),
        pltpu.VMEM((CH, H), jnp.float32),
        pltpu.VMEM_SHARED((AGGN, H), jnp.float32),
        pltpu.SemaphoreType.DMA,
        pltpu.SemaphoreType.DMA,
        pltpu.SemaphoreType.DMA,
    ],
)
def _stage_c(rel_hbm, dst_hbm, et_hbm, agg1_hbm, agg_out,
             idx_dst, idx_et, rows, rows1, agg_sh, sem_g0, sem_g1, sem_s):
    c = lax.axis_index("c")
    s = lax.axis_index("s")
    wid = s * NC + c

    pltpu.sync_copy(agg1_hbm.at[c, pl.ds(s * AGG_PT, AGG_PT), :],
                    agg_sh.at[pl.ds(s * AGG_PT, AGG_PT), :])
    plsc.subcore_barrier()

    @pl.loop(0, NCHUNK // 2)
    def _(o):
        j0 = o * 2
        pltpu.sync_copy(dst_hbm.at[wid, pl.ds(j0, 2), :], idx_dst)
        pltpu.sync_copy(et_hbm.at[wid, pl.ds(j0, 2), :], idx_et)
        g0 = pltpu.async_copy(rel_hbm.at[idx_et.at[0]], rows, sem_g0)
        g1 = pltpu.async_copy(rel_hbm.at[idx_et.at[1]], rows1, sem_g1)
        g0.wait()
        s0 = pltpu.async_copy(rows, agg_sh.at[idx_dst.at[0]], sem_s, add=True)
        g1.wait()
        s1 = pltpu.async_copy(rows1, agg_sh.at[idx_dst.at[1]], sem_s, add=True)
        s0.wait()
        s1.wait()

    jt = NCHUNK - 1
    pltpu.sync_copy(dst_hbm.at[wid, pl.ds(jt, 1), :], idx_dst.at[pl.ds(0, 1)])
    pltpu.sync_copy(et_hbm.at[wid, pl.ds(jt, 1), :], idx_et.at[pl.ds(0, 1)])
    pltpu.async_copy(rel_hbm.at[idx_et.at[0]], rows, sem_g0).wait()
    pltpu.sync_copy(rows, agg_sh.at[idx_dst.at[0]], add=True)

    plsc.subcore_barrier()
    pltpu.sync_copy(agg_sh.at[pl.ds(s * AGG_PT, AGG_PT), :],
                    agg_out.at[c, pl.ds(s * AGG_PT, AGG_PT), :])


def _gru_body(emb_rel_ref, relsum_ref, relcnt_ref, w_ih_ref, w_hh_ref,
              b_ih_ref, b_hh_ref, out_ref):
    emb_rel = emb_rel_ref[...]
    rs = relsum_ref[0] + relsum_ref[1]
    rc = jnp.sum(relcnt_ref[...], axis=1)
    x_input = rs / jnp.maximum(rc, 1.0)[:, None]
    x_cat = jnp.concatenate([emb_rel, x_input], axis=1)
    gi = lax.dot_general(x_cat, w_ih_ref[...], (((1,), (1,)), ((), ())),
                         preferred_element_type=jnp.float32) + b_ih_ref[...]
    gh = lax.dot_general(emb_rel, w_hh_ref[...], (((1,), (1,)), ((), ())),
                         preferred_element_type=jnp.float32) + b_hh_ref[...]
    r = jax.nn.sigmoid(gi[:, :H] + gh[:, :H])
    z = jax.nn.sigmoid(gi[:, H:2 * H] + gh[:, H:2 * H])
    n = jnp.tanh(gi[:, 2 * H:] + r * gh[:, 2 * H:])
    out_ref[...] = (1.0 - z) * n + z * emb_rel


_FINAL_R = 1000


def _final_body(agg2_ref, cnt_ref, h_ref, w_ref, tgw_ref, tgb_ref, out_ref):
    deg = jnp.sum(cnt_ref[...], axis=1)
    agg = (agg2_ref[0] + agg2_ref[1]) / jnp.maximum(deg, 1.0)[:, None]
    hn = lax.dot_general(agg, w_ref[...], (((1,), (0,)), ((), ())),
                         preferred_element_type=jnp.float32)
    hn = jnp.where(hn >= 0, hn, _RRELU_SLOPE * hn)
    gate = jax.nn.sigmoid(
        lax.dot_general(hn, tgw_ref[...], (((1,), (0,)), ((), ())),
                        preferred_element_type=jnp.float32) + tgb_ref[...])
    out_ref[...] = gate * hn + (1.0 - gate) * h_ref[...]


def kernel(edge_index, edge_type, dynamic_emb, emb_rel, W_ih, W_hh, b_ih, b_hh,
           rgcn_W, time_gate_weight, time_gate_bias):
    src = edge_index[0].reshape(NW, NCHUNK, CH)
    dst = edge_index[1].reshape(NW, NCHUNK, CH)
    et = edge_type.reshape(NW, NCHUNK, CH)
    agg1, relsum, cnt = _stage_a(dynamic_emb, src, dst, et)
    cnt_t = cnt.reshape(NW, CNTN).T

    rel_evolved = pl.pallas_call(
        _gru_body,
        out_shape=jax.ShapeDtypeStruct((NR2, H), jnp.float32),
    )(emb_rel, relsum[:, :NR2, :], cnt_t[NUM_ENTS:, :],
      W_ih, W_hh, b_ih.reshape(1, 3 * H), b_hh.reshape(1, 3 * H))

    agg2 = _stage_c(rel_evolved, dst, et, agg1)

    out = pl.pallas_call(
        _final_body,
        grid=(NUM_ENTS // _FINAL_R,),
        in_specs=[
            pl.BlockSpec((NC, _FINAL_R, H), lambda i: (0, i, 0)),
            pl.BlockSpec((_FINAL_R, NW), lambda i: (i, 0)),
            pl.BlockSpec((_FINAL_R, H), lambda i: (i, 0)),
            pl.BlockSpec((H, H), lambda i: (0, 0)),
            pl.BlockSpec((H, H), lambda i: (0, 0)),
            pl.BlockSpec((1, H), lambda i: (0, 0)),
        ],
        out_specs=pl.BlockSpec((_FINAL_R, H), lambda i: (i, 0)),
        out_shape=jax.ShapeDtypeStruct((NUM_ENTS, H), jnp.float32),
    )(agg2, cnt_t[:NUM_ENTS, :], dynamic_emb, rgcn_W, time_gate_weight,
      time_gate_bias.reshape(1, H))
    return out


# speedup vs baseline: 6.9196x; 2.2080x over previous
import functools

import jax
import jax.numpy as jnp
from jax import lax
from jax.experimental import pallas as pl
from jax.experimental.pallas import tpu as pltpu
from jax.experimental.pallas import tpu_sc as plsc

NUM_ENTS = 10000
H = 128
N_EDGES = 320000
NR2 = 400

NC = 2
NS = 16
NW = NC * NS
CH = 80
EPW = N_EDGES // NW
NCHUNK = EPW // CH

AGGN = 10240
AGG_PT = AGGN // NS
RELN = 512
REL_PT = RELN // NS
CNTN = 10400

_RRELU_SLOPE = (1.0 / 8.0 + 1.0 / 3.0) / 2.0

_MESH = plsc.VectorSubcoreMesh(
    core_axis_name="c", subcore_axis_name="s", num_cores=NC, num_subcores=NS)


def _zero_vmem_f32(ref, nrows, ncols):
    z = jnp.zeros((16,), jnp.float32)

    @pl.loop(0, nrows * (ncols // 16))
    def _(t):
        i = t // (ncols // 16)
        k = (t % (ncols // 16)) * 16
        ref[i, pl.ds(k, 16)] = z


@functools.partial(
    pl.kernel,
    out_type=(
        jax.ShapeDtypeStruct((NC, AGGN, H), jnp.float32),
        jax.ShapeDtypeStruct((NC, RELN, H), jnp.float32),
        jax.ShapeDtypeStruct((NC, NS, CNTN), jnp.float32),
    ),
    mesh=_MESH,
    compiler_params=pltpu.CompilerParams(use_tc_tiling_on_sc=False, needs_layout_passes=False),
    scratch_types=[
        pltpu.VMEM((2, CH), jnp.int32),
        pltpu.VMEM((2, CH), jnp.int32),
        pltpu.VMEM((2, CH), jnp.int32),
        pltpu.VMEM((CH, H), jnp.float32),
        pltpu.VMEM((CH, H), jnp.float32),
        pltpu.VMEM((CNTN,), jnp.float32),
        pltpu.VMEM_SHARED((AGGN, H), jnp.float32),
        pltpu.VMEM_SHARED((RELN, H), jnp.float32),
        pltpu.SemaphoreType.DMA,
        pltpu.SemaphoreType.DMA,
        pltpu.SemaphoreType.DMA,
    ],
)
def _stage_a(h_hbm, src_hbm, dst_hbm, et_hbm,
             agg_out, rel_out, cnt_out,
             idx_src, idx_dst, idx_et, rows, rows1, cnt_v,
             agg_sh, rel_sh, sem_g0, sem_g1, sem_s):
    c = lax.axis_index("c")
    s = lax.axis_index("s")
    wid = s * NC + c

    _zero_vmem_f32(rows, CH, H)
    for j in range(AGG_PT // CH):
        pltpu.sync_copy(rows, agg_sh.at[pl.ds(s * AGG_PT + j * CH, CH), :])
    pltpu.sync_copy(rows.at[pl.ds(0, REL_PT), :],
                    rel_sh.at[pl.ds(s * REL_PT, REL_PT), :])

    z = jnp.zeros((16,), jnp.float32)

    @pl.loop(0, CNTN // 16)
    def _(t):
        cnt_v[pl.ds(t * 16, 16)] = z

    plsc.subcore_barrier()

    ones16 = jnp.full((16,), 1.0, jnp.float32)

    def _count(buf_row):
        for v in range(CH // 16):
            d = idx_dst[buf_row, pl.ds(v * 16, 16)]
            plsc.addupdate_scatter(cnt_v, [d], ones16)
            e = idx_et[buf_row, pl.ds(v * 16, 16)] + NUM_ENTS
            plsc.addupdate_scatter(cnt_v, [e], ones16)

    @pl.loop(0, NCHUNK // 2)
    def _(o):
        j0 = o * 2
        pltpu.sync_copy(src_hbm.at[wid, pl.ds(j0, 2), :], idx_src)
        pltpu.sync_copy(dst_hbm.at[wid, pl.ds(j0, 2), :], idx_dst)
        pltpu.sync_copy(et_hbm.at[wid, pl.ds(j0, 2), :], idx_et)
        g0 = pltpu.async_copy(h_hbm.at[idx_src.at[0]], rows, sem_g0)
        g1 = pltpu.async_copy(h_hbm.at[idx_src.at[1]], rows1, sem_g1)
        g0.wait()
        pend = [
            pltpu.async_copy(rows, agg_sh.at[idx_dst.at[0]], sem_s, add=True),
            pltpu.async_copy(rows, rel_sh.at[idx_et.at[0]], sem_s, add=True),
        ]
        g1.wait()
        pend += [
            pltpu.async_copy(rows1, agg_sh.at[idx_dst.at[1]], sem_s, add=True),
            pltpu.async_copy(rows1, rel_sh.at[idx_et.at[1]], sem_s, add=True),
        ]
        _count(0)
        _count(1)
        for p in pend:
            p.wait()

    jt = NCHUNK - 1
    pltpu.sync_copy(src_hbm.at[wid, pl.ds(jt, 1), :], idx_src.at[pl.ds(0, 1)])
    pltpu.sync_copy(dst_hbm.at[wid, pl.ds(jt, 1), :], idx_dst.at[pl.ds(0, 1)])
    pltpu.sync_copy(et_hbm.at[wid, pl.ds(jt, 1), :], idx_et.at[pl.ds(0, 1)])
    pltpu.async_copy(h_hbm.at[idx_src.at[0]], rows, sem_g0).wait()
    pltpu.sync_copy(rows, agg_sh.at[idx_dst.at[0]], add=True)
    pltpu.sync_copy(rows, rel_sh.at[idx_et.at[0]], add=True)
    _count(0)

    plsc.subcore_barrier()
    pltpu.sync_copy(agg_sh.at[pl.ds(s * AGG_PT, AGG_PT), :],
                    agg_out.at[c, pl.ds(s * AGG_PT, AGG_PT), :])
    pltpu.sync_copy(rel_sh.at[pl.ds(s * REL_PT, REL_PT), :],
                    rel_out.at[c, pl.ds(s * REL_PT, REL_PT), :])
    pltpu.sync_copy(cnt_v, cnt_out.at[c, s, :])


@functools.partial(
    pl.kernel,
    out_type=jax.ShapeDtypeStruct((NC, AGGN, H), jnp.float32),
    mesh=_MESH,
    compiler_params=pltpu.CompilerParams(use_tc_tiling_on_sc=False, needs_layout_passes=False),
    scratch_types=[
        pltpu.VMEM((2, CH), jnp.int32),
        pltpu.VMEM((2, CH), jnp.int32),
        pltpu.VMEM((CH, H), jnp.float32),
        pltpu.VMEM((CH, H), jnp.float32),
        pltpu.VMEM_SHARED((AGGN, H), jnp.float32),
        pltpu.VMEM_SHARED((NR2, H), jnp.float32),
        pltpu.SemaphoreType.DMA,
        pltpu.SemaphoreType.DMA,
        pltpu.SemaphoreType.DMA,
    ],
)
def _stage_c(rel_hbm, dst_hbm, et_hbm, agg1_hbm, agg_out,
             idx_dst, idx_et, rows, rows1, agg_sh, rel_sh, sem_g0, sem_g1,
             sem_s):
    c = lax.axis_index("c")
    s = lax.axis_index("s")
    wid = s * NC + c

    pltpu.sync_copy(agg1_hbm.at[c, pl.ds(s * AGG_PT, AGG_PT), :],
                    agg_sh.at[pl.ds(s * AGG_PT, AGG_PT), :])
    nrel_pt = NR2 // NS
    pltpu.sync_copy(rel_hbm.at[pl.ds(s * nrel_pt, nrel_pt), :],
                    rel_sh.at[pl.ds(s * nrel_pt, nrel_pt), :])
    plsc.subcore_barrier()

    @pl.loop(0, NCHUNK // 2)
    def _(o):
        j0 = o * 2
        pltpu.sync_copy(dst_hbm.at[wid, pl.ds(j0, 2), :], idx_dst)
        pltpu.sync_copy(et_hbm.at[wid, pl.ds(j0, 2), :], idx_et)
        g0 = pltpu.async_copy(rel_sh.at[idx_et.at[0]], rows, sem_g0)
        g1 = pltpu.async_copy(rel_sh.at[idx_et.at[1]], rows1, sem_g1)
        g0.wait()
        s0 = pltpu.async_copy(rows, agg_sh.at[idx_dst.at[0]], sem_s, add=True)
        g1.wait()
        s1 = pltpu.async_copy(rows1, agg_sh.at[idx_dst.at[1]], sem_s, add=True)
        s0.wait()
        s1.wait()

    jt = NCHUNK - 1
    pltpu.sync_copy(dst_hbm.at[wid, pl.ds(jt, 1), :], idx_dst.at[pl.ds(0, 1)])
    pltpu.sync_copy(et_hbm.at[wid, pl.ds(jt, 1), :], idx_et.at[pl.ds(0, 1)])
    pltpu.async_copy(rel_sh.at[idx_et.at[0]], rows, sem_g0).wait()
    pltpu.sync_copy(rows, agg_sh.at[idx_dst.at[0]], add=True)

    plsc.subcore_barrier()
    pltpu.sync_copy(agg_sh.at[pl.ds(s * AGG_PT, AGG_PT), :],
                    agg_out.at[c, pl.ds(s * AGG_PT, AGG_PT), :])


def _gru_body(emb_rel_ref, relsum_ref, relcnt_ref, w_ih_ref, w_hh_ref,
              b_ih_ref, b_hh_ref, out_ref):
    emb_rel = emb_rel_ref[...]
    rs = relsum_ref[0] + relsum_ref[1]
    rc = jnp.sum(relcnt_ref[...], axis=1)
    x_input = rs / jnp.maximum(rc, 1.0)[:, None]
    x_cat = jnp.concatenate([emb_rel, x_input], axis=1)
    gi = lax.dot_general(x_cat, w_ih_ref[...], (((1,), (1,)), ((), ())),
                         preferred_element_type=jnp.float32) + b_ih_ref[...]
    gh = lax.dot_general(emb_rel, w_hh_ref[...], (((1,), (1,)), ((), ())),
                         preferred_element_type=jnp.float32) + b_hh_ref[...]
    r = jax.nn.sigmoid(gi[:, :H] + gh[:, :H])
    z = jax.nn.sigmoid(gi[:, H:2 * H] + gh[:, H:2 * H])
    n = jnp.tanh(gi[:, 2 * H:] + r * gh[:, 2 * H:])
    out_ref[...] = (1.0 - z) * n + z * emb_rel


_FINAL_R = 1000


def _final_body(agg2_ref, cnt_ref, h_ref, w_ref, tgw_ref, tgb_ref, out_ref):
    deg = jnp.sum(cnt_ref[...], axis=1)
    agg = (agg2_ref[0] + agg2_ref[1]) / jnp.maximum(deg, 1.0)[:, None]
    hn = lax.dot_general(agg, w_ref[...], (((1,), (0,)), ((), ())),
                         preferred_element_type=jnp.float32)
    hn = jnp.where(hn >= 0, hn, _RRELU_SLOPE * hn)
    gate = jax.nn.sigmoid(
        lax.dot_general(hn, tgw_ref[...], (((1,), (0,)), ((), ())),
                        preferred_element_type=jnp.float32) + tgb_ref[...])
    out_ref[...] = gate * hn + (1.0 - gate) * h_ref[...]


def kernel(edge_index, edge_type, dynamic_emb, emb_rel, W_ih, W_hh, b_ih, b_hh,
           rgcn_W, time_gate_weight, time_gate_bias):
    src = edge_index[0].reshape(NW, NCHUNK, CH)
    dst = edge_index[1].reshape(NW, NCHUNK, CH)
    et = edge_type.reshape(NW, NCHUNK, CH)
    agg1, relsum, cnt = _stage_a(dynamic_emb, src, dst, et)
    cnt_t = cnt.reshape(NW, CNTN).T

    rel_evolved = pl.pallas_call(
        _gru_body,
        out_shape=jax.ShapeDtypeStruct((NR2, H), jnp.float32),
    )(emb_rel, relsum[:, :NR2, :], cnt_t[NUM_ENTS:, :],
      W_ih, W_hh, b_ih.reshape(1, 3 * H), b_hh.reshape(1, 3 * H))

    agg2 = _stage_c(rel_evolved, dst, et, agg1)

    out = pl.pallas_call(
        _final_body,
        grid=(NUM_ENTS // _FINAL_R,),
        in_specs=[
            pl.BlockSpec((NC, _FINAL_R, H), lambda i: (0, i, 0)),
            pl.BlockSpec((_FINAL_R, NW), lambda i: (i, 0)),
            pl.BlockSpec((_FINAL_R, H), lambda i: (i, 0)),
            pl.BlockSpec((H, H), lambda i: (0, 0)),
            pl.BlockSpec((H, H), lambda i: (0, 0)),
            pl.BlockSpec((1, H), lambda i: (0, 0)),
        ],
        out_specs=pl.BlockSpec((_FINAL_R, H), lambda i: (i, 0)),
        out_shape=jax.ShapeDtypeStruct((NUM_ENTS, H), jnp.float32),
    )(agg2, cnt_t[:NUM_ENTS, :], dynamic_emb, rgcn_W, time_gate_weight,
      time_gate_bias.reshape(1, H))
    return out


# speedup vs baseline: 7.8495x; 1.1344x over previous
import functools

import jax
import jax.numpy as jnp
from jax import lax
from jax.experimental import pallas as pl
from jax.experimental.pallas import tpu as pltpu
from jax.experimental.pallas import tpu_sc as plsc

NUM_ENTS = 10000
H = 128
N_EDGES = 320000
NR2 = 400

NC = 2
NS = 16
NW = NC * NS
CH = 80
EPW = N_EDGES // NW
NCHUNK = EPW // CH
BI = 10

AGGN = 10240
AGG_PT = AGGN // NS
RELN = 512
REL_PT = RELN // NS
CNTN = 10400

_RRELU_SLOPE = (1.0 / 8.0 + 1.0 / 3.0) / 2.0

_MESH = plsc.VectorSubcoreMesh(
    core_axis_name="c", subcore_axis_name="s", num_cores=NC, num_subcores=NS)


def _zero_vmem_f32(ref, nrows, ncols):
    z = jnp.zeros((16,), jnp.float32)

    @pl.loop(0, nrows * (ncols // 16))
    def _(t):
        i = t // (ncols // 16)
        k = (t % (ncols // 16)) * 16
        ref[i, pl.ds(k, 16)] = z


@functools.partial(
    pl.kernel,
    out_type=(
        jax.ShapeDtypeStruct((NC, AGGN, H), jnp.float32),
        jax.ShapeDtypeStruct((NC, RELN, H), jnp.float32),
        jax.ShapeDtypeStruct((NC, NS, CNTN), jnp.float32),
    ),
    mesh=_MESH,
    compiler_params=pltpu.CompilerParams(use_tc_tiling_on_sc=False, needs_layout_passes=False),
    scratch_types=[
        pltpu.VMEM((BI, CH), jnp.int32),
        pltpu.VMEM((BI, CH), jnp.int32),
        pltpu.VMEM((BI, CH), jnp.int32),
        pltpu.VMEM((CH, H), jnp.float32),
        pltpu.VMEM((CH, H), jnp.float32),
        pltpu.VMEM((CNTN,), jnp.float32),
        pltpu.VMEM_SHARED((AGGN, H), jnp.float32),
        pltpu.VMEM_SHARED((RELN, H), jnp.float32),
        pltpu.SemaphoreType.DMA,
        pltpu.SemaphoreType.DMA,
        pltpu.SemaphoreType.DMA,
    ],
)
def _stage_a(h_hbm, src_hbm, dst_hbm, et_hbm,
             agg_out, rel_out, cnt_out,
             idx_src, idx_dst, idx_et, rows, rows1, cnt_v,
             agg_sh, rel_sh, sem_g0, sem_g1, sem_s):
    c = lax.axis_index("c")
    s = lax.axis_index("s")
    wid = s * NC + c

    _zero_vmem_f32(rows, CH, H)
    for j in range(AGG_PT // CH):
        pltpu.sync_copy(rows, agg_sh.at[pl.ds(s * AGG_PT + j * CH, CH), :])
    pltpu.sync_copy(rows.at[pl.ds(0, REL_PT), :],
                    rel_sh.at[pl.ds(s * REL_PT, REL_PT), :])

    z = jnp.zeros((16,), jnp.float32)

    @pl.loop(0, CNTN // 16)
    def _(t):
        cnt_v[pl.ds(t * 16, 16)] = z

    plsc.subcore_barrier()

    ones16 = jnp.full((16,), 1.0, jnp.float32)

    def _count(buf_row):
        for v in range(CH // 16):
            d = idx_dst[buf_row, pl.ds(v * 16, 16)]
            plsc.addupdate_scatter(cnt_v, [d], ones16)
            e = idx_et[buf_row, pl.ds(v * 16, 16)] + NUM_ENTS
            plsc.addupdate_scatter(cnt_v, [e], ones16)

    def _pair(b0, b1):
        g0 = pltpu.async_copy(h_hbm.at[idx_src.at[b0]], rows, sem_g0)
        g1 = pltpu.async_copy(h_hbm.at[idx_src.at[b1]], rows1, sem_g1)
        g0.wait()
        pend = [
            pltpu.async_copy(rows, agg_sh.at[idx_dst.at[b0]], sem_s, add=True),
            pltpu.async_copy(rows, rel_sh.at[idx_et.at[b0]], sem_s, add=True),
        ]
        g1.wait()
        pend += [
            pltpu.async_copy(rows1, agg_sh.at[idx_dst.at[b1]], sem_s, add=True),
            pltpu.async_copy(rows1, rel_sh.at[idx_et.at[b1]], sem_s, add=True),
        ]
        _count(b0)
        _count(b1)
        for p in pend:
            p.wait()

    @pl.loop(0, NCHUNK // BI)
    def _(o):
        j0 = o * BI
        pltpu.sync_copy(src_hbm.at[wid, pl.ds(j0, BI), :], idx_src)
        pltpu.sync_copy(dst_hbm.at[wid, pl.ds(j0, BI), :], idx_dst)
        pltpu.sync_copy(et_hbm.at[wid, pl.ds(j0, BI), :], idx_et)
        for b in range(BI // 2):
            _pair(2 * b, 2 * b + 1)

    jt = (NCHUNK // BI) * BI
    pltpu.sync_copy(src_hbm.at[wid, pl.ds(jt, NCHUNK - jt), :],
                    idx_src.at[pl.ds(0, NCHUNK - jt)])
    pltpu.sync_copy(dst_hbm.at[wid, pl.ds(jt, NCHUNK - jt), :],
                    idx_dst.at[pl.ds(0, NCHUNK - jt)])
    pltpu.sync_copy(et_hbm.at[wid, pl.ds(jt, NCHUNK - jt), :],
                    idx_et.at[pl.ds(0, NCHUNK - jt)])
    for b in range((NCHUNK - jt) // 2):
        _pair(2 * b, 2 * b + 1)
    bl = NCHUNK - jt - 1
    pltpu.async_copy(h_hbm.at[idx_src.at[bl]], rows, sem_g0).wait()
    pltpu.sync_copy(rows, agg_sh.at[idx_dst.at[bl]], add=True)
    pltpu.sync_copy(rows, rel_sh.at[idx_et.at[bl]], add=True)
    _count(bl)

    plsc.subcore_barrier()
    pltpu.sync_copy(agg_sh.at[pl.ds(s * AGG_PT, AGG_PT), :],
                    agg_out.at[c, pl.ds(s * AGG_PT, AGG_PT), :])
    pltpu.sync_copy(rel_sh.at[pl.ds(s * REL_PT, REL_PT), :],
                    rel_out.at[c, pl.ds(s * REL_PT, REL_PT), :])
    pltpu.sync_copy(cnt_v, cnt_out.at[c, s, :])


@functools.partial(
    pl.kernel,
    out_type=jax.ShapeDtypeStruct((NC, AGGN, H), jnp.float32),
    mesh=_MESH,
    compiler_params=pltpu.CompilerParams(use_tc_tiling_on_sc=False, needs_layout_passes=False),
    scratch_types=[
        pltpu.VMEM((BI, CH), jnp.int32),
        pltpu.VMEM((BI, CH), jnp.int32),
        pltpu.VMEM((CH, H), jnp.float32),
        pltpu.VMEM((CH, H), jnp.float32),
        pltpu.VMEM_SHARED((AGGN, H), jnp.float32),
        pltpu.VMEM_SHARED((NR2, H), jnp.float32),
        pltpu.SemaphoreType.DMA,
        pltpu.SemaphoreType.DMA,
        pltpu.SemaphoreType.DMA,
    ],
)
def _stage_c(rel_hbm, dst_hbm, et_hbm, agg1_hbm, agg_out,
             idx_dst, idx_et, rows, rows1, agg_sh, rel_sh, sem_g0, sem_g1,
             sem_s):
    c = lax.axis_index("c")
    s = lax.axis_index("s")
    wid = s * NC + c

    pltpu.sync_copy(agg1_hbm.at[c, pl.ds(s * AGG_PT, AGG_PT), :],
                    agg_sh.at[pl.ds(s * AGG_PT, AGG_PT), :])
    nrel_pt = NR2 // NS
    pltpu.sync_copy(rel_hbm.at[pl.ds(s * nrel_pt, nrel_pt), :],
                    rel_sh.at[pl.ds(s * nrel_pt, nrel_pt), :])
    plsc.subcore_barrier()

    def _pair(b0, b1):
        g0 = pltpu.async_copy(rel_sh.at[idx_et.at[b0]], rows, sem_g0)
        g1 = pltpu.async_copy(rel_sh.at[idx_et.at[b1]], rows1, sem_g1)
        g0.wait()
        s0 = pltpu.async_copy(rows, agg_sh.at[idx_dst.at[b0]], sem_s, add=True)
        g1.wait()
        s1 = pltpu.async_copy(rows1, agg_sh.at[idx_dst.at[b1]], sem_s, add=True)
        s0.wait()
        s1.wait()

    @pl.loop(0, NCHUNK // BI)
    def _(o):
        j0 = o * BI
        pltpu.sync_copy(dst_hbm.at[wid, pl.ds(j0, BI), :], idx_dst)
        pltpu.sync_copy(et_hbm.at[wid, pl.ds(j0, BI), :], idx_et)
        for b in range(BI // 2):
            _pair(2 * b, 2 * b + 1)

    jt = (NCHUNK // BI) * BI
    pltpu.sync_copy(dst_hbm.at[wid, pl.ds(jt, NCHUNK - jt), :],
                    idx_dst.at[pl.ds(0, NCHUNK - jt)])
    pltpu.sync_copy(et_hbm.at[wid, pl.ds(jt, NCHUNK - jt), :],
                    idx_et.at[pl.ds(0, NCHUNK - jt)])
    for b in range((NCHUNK - jt) // 2):
        _pair(2 * b, 2 * b + 1)
    bl = NCHUNK - jt - 1
    pltpu.async_copy(rel_sh.at[idx_et.at[bl]], rows, sem_g0).wait()
    pltpu.sync_copy(rows, agg_sh.at[idx_dst.at[bl]], add=True)

    plsc.subcore_barrier()
    pltpu.sync_copy(agg_sh.at[pl.ds(s * AGG_PT, AGG_PT), :],
                    agg_out.at[c, pl.ds(s * AGG_PT, AGG_PT), :])


def _gru_body(emb_rel_ref, relsum_ref, relcnt_ref, w_ih_ref, w_hh_ref,
              b_ih_ref, b_hh_ref, out_ref):
    emb_rel = emb_rel_ref[...]
    rs = relsum_ref[0] + relsum_ref[1]
    rc = jnp.sum(relcnt_ref[...], axis=1)
    x_input = rs / jnp.maximum(rc, 1.0)[:, None]
    x_cat = jnp.concatenate([emb_rel, x_input], axis=1)
    gi = lax.dot_general(x_cat, w_ih_ref[...], (((1,), (1,)), ((), ())),
                         preferred_element_type=jnp.float32) + b_ih_ref[...]
    gh = lax.dot_general(emb_rel, w_hh_ref[...], (((1,), (1,)), ((), ())),
                         preferred_element_type=jnp.float32) + b_hh_ref[...]
    r = jax.nn.sigmoid(gi[:, :H] + gh[:, :H])
    z = jax.nn.sigmoid(gi[:, H:2 * H] + gh[:, H:2 * H])
    n = jnp.tanh(gi[:, 2 * H:] + r * gh[:, 2 * H:])
    out_ref[...] = (1.0 - z) * n + z * emb_rel


_FINAL_R = 1000


def _final_body(agg2_ref, cnt_ref, h_ref, w_ref, tgw_ref, tgb_ref, out_ref):
    deg = jnp.sum(cnt_ref[...], axis=1)
    agg = (agg2_ref[0] + agg2_ref[1]) / jnp.maximum(deg, 1.0)[:, None]
    hn = lax.dot_general(agg, w_ref[...], (((1,), (0,)), ((), ())),
                         preferred_element_type=jnp.float32)
    hn = jnp.where(hn >= 0, hn, _RRELU_SLOPE * hn)
    gate = jax.nn.sigmoid(
        lax.dot_general(hn, tgw_ref[...], (((1,), (0,)), ((), ())),
                        preferred_element_type=jnp.float32) + tgb_ref[...])
    out_ref[...] = gate * hn + (1.0 - gate) * h_ref[...]


def kernel(edge_index, edge_type, dynamic_emb, emb_rel, W_ih, W_hh, b_ih, b_hh,
           rgcn_W, time_gate_weight, time_gate_bias):
    src = edge_index[0].reshape(NW, NCHUNK, CH)
    dst = edge_index[1].reshape(NW, NCHUNK, CH)
    et = edge_type.reshape(NW, NCHUNK, CH)
    agg1, relsum, cnt = _stage_a(dynamic_emb, src, dst, et)
    cnt_t = cnt.reshape(NW, CNTN).T

    rel_evolved = pl.pallas_call(
        _gru_body,
        out_shape=jax.ShapeDtypeStruct((NR2, H), jnp.float32),
    )(emb_rel, relsum[:, :NR2, :], cnt_t[NUM_ENTS:, :],
      W_ih, W_hh, b_ih.reshape(1, 3 * H), b_hh.reshape(1, 3 * H))

    agg2 = _stage_c(rel_evolved, dst, et, agg1)

    out = pl.pallas_call(
        _final_body,
        grid=(NUM_ENTS // _FINAL_R,),
        in_specs=[
            pl.BlockSpec((NC, _FINAL_R, H), lambda i: (0, i, 0)),
            pl.BlockSpec((_FINAL_R, NW), lambda i: (i, 0)),
            pl.BlockSpec((_FINAL_R, H), lambda i: (i, 0)),
            pl.BlockSpec((H, H), lambda i: (0, 0)),
            pl.BlockSpec((H, H), lambda i: (0, 0)),
            pl.BlockSpec((1, H), lambda i: (0, 0)),
        ],
        out_specs=pl.BlockSpec((_FINAL_R, H), lambda i: (i, 0)),
        out_shape=jax.ShapeDtypeStruct((NUM_ENTS, H), jnp.float32),
    )(agg2, cnt_t[:NUM_ENTS, :], dynamic_emb, rgcn_W, time_gate_weight,
      time_gate_bias.reshape(1, H))
    return out


# speedup vs baseline: 8.8213x; 1.1238x over previous
import functools

import jax
import jax.numpy as jnp
from jax import lax
from jax.experimental import pallas as pl
from jax.experimental.pallas import tpu as pltpu
from jax.experimental.pallas import tpu_sc as plsc

NUM_ENTS = 10000
H = 128
N_EDGES = 320000
NR2 = 400

NC = 2
NS = 16
NW = NC * NS
CH = 80
EPW = N_EDGES // NW
NCHUNK = EPW // CH
BI = 10

AGGN = 10240
AGG_PT = AGGN // NS
RELN = 512
REL_PT = RELN // NS
CNTN = 10400

_RRELU_SLOPE = (1.0 / 8.0 + 1.0 / 3.0) / 2.0

_MESH = plsc.VectorSubcoreMesh(
    core_axis_name="c", subcore_axis_name="s", num_cores=NC, num_subcores=NS)


def _zero_vmem_f32(ref, nrows, ncols):
    z = jnp.zeros((16,), jnp.float32)

    @pl.loop(0, nrows * (ncols // 16))
    def _(t):
        i = t // (ncols // 16)
        k = (t % (ncols // 16)) * 16
        ref[i, pl.ds(k, 16)] = z


@functools.partial(
    pl.kernel,
    out_type=(
        jax.ShapeDtypeStruct((NC, AGGN, H), jnp.float32),
        jax.ShapeDtypeStruct((NC, RELN, H), jnp.float32),
        jax.ShapeDtypeStruct((NC, NS, CNTN), jnp.float32),
    ),
    mesh=_MESH,
    compiler_params=pltpu.CompilerParams(use_tc_tiling_on_sc=False, needs_layout_passes=False),
    scratch_types=[
        pltpu.VMEM((BI, CH), jnp.int32),
        pltpu.VMEM((BI, CH), jnp.int32),
        pltpu.VMEM((BI, CH), jnp.int32),
        pltpu.VMEM((CH, H), jnp.float32),
        pltpu.VMEM((CH, H), jnp.float32),
        pltpu.VMEM((CH, H), jnp.float32),
        pltpu.VMEM((CNTN,), jnp.float32),
        pltpu.VMEM_SHARED((AGGN, H), jnp.float32),
        pltpu.VMEM_SHARED((RELN, H), jnp.float32),
        pltpu.SemaphoreType.DMA,
        pltpu.SemaphoreType.DMA,
        pltpu.SemaphoreType.DMA,
        pltpu.SemaphoreType.DMA,
        pltpu.SemaphoreType.DMA,
        pltpu.SemaphoreType.DMA,
    ],
)
def _stage_a(h_hbm, src_hbm, dst_hbm, et_hbm,
             agg_out, rel_out, cnt_out,
             idx_src, idx_dst, idx_et, rows, rows1, rows2, cnt_v,
             agg_sh, rel_sh, sem_g0, sem_g1, sem_g2,
             sem_s0, sem_s1, sem_s2):
    c = lax.axis_index("c")
    s = lax.axis_index("s")
    wid = s * NC + c

    _zero_vmem_f32(rows, CH, H)
    for j in range(AGG_PT // CH):
        pltpu.sync_copy(rows, agg_sh.at[pl.ds(s * AGG_PT + j * CH, CH), :])
    pltpu.sync_copy(rows.at[pl.ds(0, REL_PT), :],
                    rel_sh.at[pl.ds(s * REL_PT, REL_PT), :])

    z = jnp.zeros((16,), jnp.float32)

    @pl.loop(0, CNTN // 16)
    def _(t):
        cnt_v[pl.ds(t * 16, 16)] = z

    plsc.subcore_barrier()

    ones16 = jnp.full((16,), 1.0, jnp.float32)

    def _count(buf_row):
        for v in range(CH // 16):
            d = idx_dst[buf_row, pl.ds(v * 16, 16)]
            plsc.addupdate_scatter(cnt_v, [d], ones16)
            e = idx_et[buf_row, pl.ds(v * 16, 16)] + NUM_ENTS
            plsc.addupdate_scatter(cnt_v, [e], ones16)

    rbufs = (rows, rows1, rows2)
    gsems = (sem_g0, sem_g1, sem_g2)
    ssems = (sem_s0, sem_s1, sem_s2)

    def _batch(nch):
        gath, scat = {}, {}
        for k in range(nch + 2):
            if k < nch:
                b = k % 3
                for p in scat.pop(b, ()):
                    p.wait()
                gath[b] = pltpu.async_copy(h_hbm.at[idx_src.at[k]],
                                           rbufs[b], gsems[b])
            if k >= 2:
                j = k - 2
                b = j % 3
                gath.pop(b).wait()
                scat[b] = (
                    pltpu.async_copy(rbufs[b], agg_sh.at[idx_dst.at[j]],
                                     ssems[b], add=True),
                    pltpu.async_copy(rbufs[b], rel_sh.at[idx_et.at[j]],
                                     ssems[b], add=True),
                )
                _count(j)
        for b in scat:
            for p in scat[b]:
                p.wait()

    @pl.loop(0, NCHUNK // BI)
    def _(o):
        j0 = o * BI
        pltpu.sync_copy(src_hbm.at[wid, pl.ds(j0, BI), :], idx_src)
        pltpu.sync_copy(dst_hbm.at[wid, pl.ds(j0, BI), :], idx_dst)
        pltpu.sync_copy(et_hbm.at[wid, pl.ds(j0, BI), :], idx_et)
        _batch(BI)

    jt = (NCHUNK // BI) * BI
    pltpu.sync_copy(src_hbm.at[wid, pl.ds(jt, NCHUNK - jt), :],
                    idx_src.at[pl.ds(0, NCHUNK - jt)])
    pltpu.sync_copy(dst_hbm.at[wid, pl.ds(jt, NCHUNK - jt), :],
                    idx_dst.at[pl.ds(0, NCHUNK - jt)])
    pltpu.sync_copy(et_hbm.at[wid, pl.ds(jt, NCHUNK - jt), :],
                    idx_et.at[pl.ds(0, NCHUNK - jt)])
    _batch(NCHUNK - jt)

    plsc.subcore_barrier()
    pltpu.sync_copy(agg_sh.at[pl.ds(s * AGG_PT, AGG_PT), :],
                    agg_out.at[c, pl.ds(s * AGG_PT, AGG_PT), :])
    pltpu.sync_copy(rel_sh.at[pl.ds(s * REL_PT, REL_PT), :],
                    rel_out.at[c, pl.ds(s * REL_PT, REL_PT), :])
    pltpu.sync_copy(cnt_v, cnt_out.at[c, s, :])


@functools.partial(
    pl.kernel,
    out_type=jax.ShapeDtypeStruct((NC, AGGN, H), jnp.float32),
    mesh=_MESH,
    compiler_params=pltpu.CompilerParams(use_tc_tiling_on_sc=False, needs_layout_passes=False),
    scratch_types=[
        pltpu.VMEM((BI, CH), jnp.int32),
        pltpu.VMEM((BI, CH), jnp.int32),
        pltpu.VMEM((CH, H), jnp.float32),
        pltpu.VMEM((CH, H), jnp.float32),
        pltpu.VMEM((CH, H), jnp.float32),
        pltpu.VMEM_SHARED((AGGN, H), jnp.float32),
        pltpu.VMEM_SHARED((NR2, H), jnp.float32),
        pltpu.SemaphoreType.DMA,
        pltpu.SemaphoreType.DMA,
        pltpu.SemaphoreType.DMA,
        pltpu.SemaphoreType.DMA,
        pltpu.SemaphoreType.DMA,
        pltpu.SemaphoreType.DMA,
    ],
)
def _stage_c(rel_hbm, dst_hbm, et_hbm, agg1_hbm, agg_out,
             idx_dst, idx_et, rows, rows1, rows2, agg_sh, rel_sh,
             sem_g0, sem_g1, sem_g2, sem_s0, sem_s1, sem_s2):
    c = lax.axis_index("c")
    s = lax.axis_index("s")
    wid = s * NC + c

    pltpu.sync_copy(agg1_hbm.at[c, pl.ds(s * AGG_PT, AGG_PT), :],
                    agg_sh.at[pl.ds(s * AGG_PT, AGG_PT), :])
    nrel_pt = NR2 // NS
    pltpu.sync_copy(rel_hbm.at[pl.ds(s * nrel_pt, nrel_pt), :],
                    rel_sh.at[pl.ds(s * nrel_pt, nrel_pt), :])
    plsc.subcore_barrier()

    rbufs = (rows, rows1, rows2)
    gsems = (sem_g0, sem_g1, sem_g2)
    ssems = (sem_s0, sem_s1, sem_s2)

    def _batch(nch):
        gath, scat = {}, {}
        for k in range(nch + 2):
            if k < nch:
                b = k % 3
                if b in scat:
                    scat.pop(b).wait()
                gath[b] = pltpu.async_copy(rel_sh.at[idx_et.at[k]],
                                           rbufs[b], gsems[b])
            if k >= 2:
                j = k - 2
                b = j % 3
                gath.pop(b).wait()
                scat[b] = pltpu.async_copy(rbufs[b], agg_sh.at[idx_dst.at[j]],
                                           ssems[b], add=True)
        for b in scat:
            scat[b].wait()

    @pl.loop(0, NCHUNK // BI)
    def _(o):
        j0 = o * BI
        pltpu.sync_copy(dst_hbm.at[wid, pl.ds(j0, BI), :], idx_dst)
        pltpu.sync_copy(et_hbm.at[wid, pl.ds(j0, BI), :], idx_et)
        _batch(BI)

    jt = (NCHUNK // BI) * BI
    pltpu.sync_copy(dst_hbm.at[wid, pl.ds(jt, NCHUNK - jt), :],
                    idx_dst.at[pl.ds(0, NCHUNK - jt)])
    pltpu.sync_copy(et_hbm.at[wid, pl.ds(jt, NCHUNK - jt), :],
                    idx_et.at[pl.ds(0, NCHUNK - jt)])
    _batch(NCHUNK - jt)

    plsc.subcore_barrier()
    pltpu.sync_copy(agg_sh.at[pl.ds(s * AGG_PT, AGG_PT), :],
                    agg_out.at[c, pl.ds(s * AGG_PT, AGG_PT), :])


def _gru_body(emb_rel_ref, relsum_ref, relcnt_ref, w_ih_ref, w_hh_ref,
              b_ih_ref, b_hh_ref, out_ref):
    emb_rel = emb_rel_ref[...]
    rs = relsum_ref[0] + relsum_ref[1]
    rc = jnp.sum(relcnt_ref[...], axis=1)
    x_input = rs / jnp.maximum(rc, 1.0)[:, None]
    x_cat = jnp.concatenate([emb_rel, x_input], axis=1)
    gi = lax.dot_general(x_cat, w_ih_ref[...], (((1,), (1,)), ((), ())),
                         preferred_element_type=jnp.float32) + b_ih_ref[...]
    gh = lax.dot_general(emb_rel, w_hh_ref[...], (((1,), (1,)), ((), ())),
                         preferred_element_type=jnp.float32) + b_hh_ref[...]
    r = jax.nn.sigmoid(gi[:, :H] + gh[:, :H])
    z = jax.nn.sigmoid(gi[:, H:2 * H] + gh[:, H:2 * H])
    n = jnp.tanh(gi[:, 2 * H:] + r * gh[:, 2 * H:])
    out_ref[...] = (1.0 - z) * n + z * emb_rel


_FINAL_R = 1000


def _final_body(agg2_ref, cnt_ref, h_ref, w_ref, tgw_ref, tgb_ref, out_ref):
    deg = jnp.sum(cnt_ref[...], axis=1)
    agg = (agg2_ref[0] + agg2_ref[1]) / jnp.maximum(deg, 1.0)[:, None]
    hn = lax.dot_general(agg, w_ref[...], (((1,), (0,)), ((), ())),
                         preferred_element_type=jnp.float32)
    hn = jnp.where(hn >= 0, hn, _RRELU_SLOPE * hn)
    gate = jax.nn.sigmoid(
        lax.dot_general(hn, tgw_ref[...], (((1,), (0,)), ((), ())),
                        preferred_element_type=jnp.float32) + tgb_ref[...])
    out_ref[...] = gate * hn + (1.0 - gate) * h_ref[...]


def kernel(edge_index, edge_type, dynamic_emb, emb_rel, W_ih, W_hh, b_ih, b_hh,
           rgcn_W, time_gate_weight, time_gate_bias):
    src = edge_index[0].reshape(NW, NCHUNK, CH)
    dst = edge_index[1].reshape(NW, NCHUNK, CH)
    et = edge_type.reshape(NW, NCHUNK, CH)
    agg1, relsum, cnt = _stage_a(dynamic_emb, src, dst, et)
    cnt_t = cnt.reshape(NW, CNTN).T

    rel_evolved = pl.pallas_call(
        _gru_body,
        out_shape=jax.ShapeDtypeStruct((NR2, H), jnp.float32),
    )(emb_rel, relsum[:, :NR2, :], cnt_t[NUM_ENTS:, :],
      W_ih, W_hh, b_ih.reshape(1, 3 * H), b_hh.reshape(1, 3 * H))

    agg2 = _stage_c(rel_evolved, dst, et, agg1)

    out = pl.pallas_call(
        _final_body,
        grid=(NUM_ENTS // _FINAL_R,),
        in_specs=[
            pl.BlockSpec((NC, _FINAL_R, H), lambda i: (0, i, 0)),
            pl.BlockSpec((_FINAL_R, NW), lambda i: (i, 0)),
            pl.BlockSpec((_FINAL_R, H), lambda i: (i, 0)),
            pl.BlockSpec((H, H), lambda i: (0, 0)),
            pl.BlockSpec((H, H), lambda i: (0, 0)),
            pl.BlockSpec((1, H), lambda i: (0, 0)),
        ],
        out_specs=pl.BlockSpec((_FINAL_R, H), lambda i: (i, 0)),
        out_shape=jax.ShapeDtypeStruct((NUM_ENTS, H), jnp.float32),
    )(agg2, cnt_t[:NUM_ENTS, :], dynamic_emb, rgcn_W, time_gate_weight,
      time_gate_bias.reshape(1, H))
    return out


# speedup vs baseline: 9.8281x; 1.1141x over previous
import functools

import jax
import jax.numpy as jnp
from jax import lax
from jax.experimental import pallas as pl
from jax.experimental.pallas import tpu as pltpu
from jax.experimental.pallas import tpu_sc as plsc

NUM_ENTS = 10000
H = 128
N_EDGES = 320000
NR2 = 400

NC = 2
NS = 16
NW = NC * NS
CH = 80
EPW = N_EDGES // NW
NCHUNK = EPW // CH
BI = 10

AGGN = 10240
AGG_PT = AGGN // NS
RELN = 512
REL_PT = RELN // NS
CNTN = 10400

_RRELU_SLOPE = (1.0 / 8.0 + 1.0 / 3.0) / 2.0

_MESH = plsc.VectorSubcoreMesh(
    core_axis_name="c", subcore_axis_name="s", num_cores=NC, num_subcores=NS)


def _zero_vmem_f32(ref, nrows, ncols):
    z = jnp.zeros((16,), jnp.float32)

    @pl.loop(0, nrows * (ncols // 16))
    def _(t):
        i = t // (ncols // 16)
        k = (t % (ncols // 16)) * 16
        ref[i, pl.ds(k, 16)] = z


@functools.partial(
    pl.kernel,
    out_type=(
        jax.ShapeDtypeStruct((NC, AGGN, H), jnp.float32),
        jax.ShapeDtypeStruct((NC, RELN, H), jnp.float32),
        jax.ShapeDtypeStruct((NC, NS, CNTN), jnp.float32),
    ),
    mesh=_MESH,
    compiler_params=pltpu.CompilerParams(use_tc_tiling_on_sc=False, needs_layout_passes=False),
    scratch_types=[
        pltpu.VMEM((BI, CH), jnp.int32),
        pltpu.VMEM((BI, CH), jnp.int32),
        pltpu.VMEM((BI, CH), jnp.int32),
        pltpu.VMEM((CH, H), jnp.float32),
        pltpu.VMEM((CH, H), jnp.float32),
        pltpu.VMEM((CH, H), jnp.float32),
        pltpu.VMEM((CNTN,), jnp.float32),
        pltpu.VMEM_SHARED((AGGN, H), jnp.float32),
        pltpu.VMEM_SHARED((RELN, H), jnp.float32),
        pltpu.SemaphoreType.DMA,
        pltpu.SemaphoreType.DMA,
        pltpu.SemaphoreType.DMA,
        pltpu.SemaphoreType.DMA,
        pltpu.SemaphoreType.DMA,
        pltpu.SemaphoreType.DMA,
    ],
)
def _stage_a(h_hbm, src_hbm, dst_hbm, et_hbm,
             agg_out, rel_out, cnt_out,
             idx_src, idx_dst, idx_et, rows, rows1, rows2, cnt_v,
             agg_sh, rel_sh, sem_g0, sem_g1, sem_g2,
             sem_s0, sem_s1, sem_s2):
    c = lax.axis_index("c")
    s = lax.axis_index("s")
    wid = s * NC + c

    _zero_vmem_f32(rows, CH, H)
    for j in range(AGG_PT // CH):
        pltpu.sync_copy(rows, agg_sh.at[pl.ds(s * AGG_PT + j * CH, CH), :])
    pltpu.sync_copy(rows.at[pl.ds(0, REL_PT), :],
                    rel_sh.at[pl.ds(s * REL_PT, REL_PT), :])

    z = jnp.zeros((16,), jnp.float32)

    @pl.loop(0, CNTN // 16)
    def _(t):
        cnt_v[pl.ds(t * 16, 16)] = z

    plsc.subcore_barrier()

    ones16 = jnp.full((16,), 1.0, jnp.float32)

    def _count(buf_row):
        for v in range(CH // 16):
            d = idx_dst[buf_row, pl.ds(v * 16, 16)]
            plsc.addupdate_scatter(cnt_v, [d], ones16)
            e = idx_et[buf_row, pl.ds(v * 16, 16)] + NUM_ENTS
            plsc.addupdate_scatter(cnt_v, [e], ones16)

    rbufs = (rows, rows1, rows2)
    gsems = (sem_g0, sem_g1, sem_g2)
    ssems = (sem_s0, sem_s1, sem_s2)

    def _batch(nch):
        gath, scat = {}, {}
        for k in range(nch + 2):
            if k < nch:
                b = k % 3
                for p in scat.pop(b, ()):
                    p.wait()
                gath[b] = pltpu.async_copy(h_hbm.at[idx_src.at[k]],
                                           rbufs[b], gsems[b])
            if k >= 2:
                j = k - 2
                b = j % 3
                gath.pop(b).wait()
                scat[b] = (
                    pltpu.async_copy(rbufs[b], agg_sh.at[idx_dst.at[j]],
                                     ssems[b], add=True),
                    pltpu.async_copy(rbufs[b], rel_sh.at[idx_et.at[j]],
                                     ssems[b], add=True),
                )
                _count(j)
        for b in scat:
            for p in scat[b]:
                p.wait()

    @pl.loop(0, NCHUNK // BI)
    def _(o):
        j0 = o * BI
        pltpu.sync_copy(src_hbm.at[wid, pl.ds(j0, BI), :], idx_src)
        pltpu.sync_copy(dst_hbm.at[wid, pl.ds(j0, BI), :], idx_dst)
        pltpu.sync_copy(et_hbm.at[wid, pl.ds(j0, BI), :], idx_et)
        _batch(BI)

    jt = (NCHUNK // BI) * BI
    pltpu.sync_copy(src_hbm.at[wid, pl.ds(jt, NCHUNK - jt), :],
                    idx_src.at[pl.ds(0, NCHUNK - jt)])
    pltpu.sync_copy(dst_hbm.at[wid, pl.ds(jt, NCHUNK - jt), :],
                    idx_dst.at[pl.ds(0, NCHUNK - jt)])
    pltpu.sync_copy(et_hbm.at[wid, pl.ds(jt, NCHUNK - jt), :],
                    idx_et.at[pl.ds(0, NCHUNK - jt)])
    _batch(NCHUNK - jt)

    plsc.subcore_barrier()
    pltpu.sync_copy(agg_sh.at[pl.ds(s * AGG_PT, AGG_PT), :],
                    agg_out.at[c, pl.ds(s * AGG_PT, AGG_PT), :])
    pltpu.sync_copy(rel_sh.at[pl.ds(s * REL_PT, REL_PT), :],
                    rel_out.at[c, pl.ds(s * REL_PT, REL_PT), :])
    pltpu.sync_copy(cnt_v, cnt_out.at[c, s, :])


@functools.partial(
    pl.kernel,
    out_type=jax.ShapeDtypeStruct((NC, AGGN, H), jnp.float32),
    mesh=_MESH,
    compiler_params=pltpu.CompilerParams(use_tc_tiling_on_sc=False, needs_layout_passes=False),
    scratch_types=[
        pltpu.VMEM((BI, CH), jnp.int32),
        pltpu.VMEM((BI, CH), jnp.int32),
        pltpu.VMEM((CH, H), jnp.float32),
        pltpu.VMEM((CH, H), jnp.float32),
        pltpu.VMEM((CH, H), jnp.float32),
        pltpu.VMEM((CH, H), jnp.float32),
        pltpu.VMEM_SHARED((AGGN, H), jnp.float32),
        pltpu.VMEM_SHARED((NR2, H), jnp.float32),
        pltpu.SemaphoreType.DMA,
        pltpu.SemaphoreType.DMA,
        pltpu.SemaphoreType.DMA,
        pltpu.SemaphoreType.DMA,
        pltpu.SemaphoreType.DMA,
        pltpu.SemaphoreType.DMA,
        pltpu.SemaphoreType.DMA,
        pltpu.SemaphoreType.DMA,
    ],
)
def _stage_c(rel_hbm, dst_hbm, et_hbm, agg1_hbm, agg_out,
             idx_dst, idx_et, rows, rows1, rows2, rows3, agg_sh, rel_sh,
             sem_g0, sem_g1, sem_g2, sem_g3, sem_s0, sem_s1, sem_s2, sem_s3):
    c = lax.axis_index("c")
    s = lax.axis_index("s")
    wid = s * NC + c

    pltpu.sync_copy(agg1_hbm.at[c, pl.ds(s * AGG_PT, AGG_PT), :],
                    agg_sh.at[pl.ds(s * AGG_PT, AGG_PT), :])
    nrel_pt = NR2 // NS
    pltpu.sync_copy(rel_hbm.at[pl.ds(s * nrel_pt, nrel_pt), :],
                    rel_sh.at[pl.ds(s * nrel_pt, nrel_pt), :])
    plsc.subcore_barrier()

    rbufs = (rows, rows1, rows2, rows3)
    gsems = (sem_g0, sem_g1, sem_g2, sem_g3)
    ssems = (sem_s0, sem_s1, sem_s2, sem_s3)

    def _batch(nch):
        gath, scat = {}, {}
        for k in range(nch + 2):
            if k < nch:
                b = k % 4
                if b in scat:
                    scat.pop(b).wait()
                gath[b] = pltpu.async_copy(rel_sh.at[idx_et.at[k]],
                                           rbufs[b], gsems[b])
            if k >= 2:
                j = k - 2
                b = j % 4
                gath.pop(b).wait()
                scat[b] = pltpu.async_copy(rbufs[b], agg_sh.at[idx_dst.at[j]],
                                           ssems[b], add=True)
        for b in scat:
            scat[b].wait()

    @pl.loop(0, NCHUNK // BI)
    def _(o):
        j0 = o * BI
        pltpu.sync_copy(dst_hbm.at[wid, pl.ds(j0, BI), :], idx_dst)
        pltpu.sync_copy(et_hbm.at[wid, pl.ds(j0, BI), :], idx_et)
        _batch(BI)

    jt = (NCHUNK // BI) * BI
    pltpu.sync_copy(dst_hbm.at[wid, pl.ds(jt, NCHUNK - jt), :],
                    idx_dst.at[pl.ds(0, NCHUNK - jt)])
    pltpu.sync_copy(et_hbm.at[wid, pl.ds(jt, NCHUNK - jt), :],
                    idx_et.at[pl.ds(0, NCHUNK - jt)])
    _batch(NCHUNK - jt)

    plsc.subcore_barrier()
    pltpu.sync_copy(agg_sh.at[pl.ds(s * AGG_PT, AGG_PT), :],
                    agg_out.at[c, pl.ds(s * AGG_PT, AGG_PT), :])


def _gru_body(emb_rel_ref, relsum_ref, relcnt_ref, w_ih_ref, w_hh_ref,
              b_ih_ref, b_hh_ref, out_ref):
    emb_rel = emb_rel_ref[...]
    rs = relsum_ref[0] + relsum_ref[1]
    rc = jnp.sum(relcnt_ref[...], axis=1)
    x_input = rs / jnp.maximum(rc, 1.0)[:, None]
    x_cat = jnp.concatenate([emb_rel, x_input], axis=1)
    gi = lax.dot_general(x_cat, w_ih_ref[...], (((1,), (1,)), ((), ())),
                         preferred_element_type=jnp.float32) + b_ih_ref[...]
    gh = lax.dot_general(emb_rel, w_hh_ref[...], (((1,), (1,)), ((), ())),
                         preferred_element_type=jnp.float32) + b_hh_ref[...]
    r = jax.nn.sigmoid(gi[:, :H] + gh[:, :H])
    z = jax.nn.sigmoid(gi[:, H:2 * H] + gh[:, H:2 * H])
    n = jnp.tanh(gi[:, 2 * H:] + r * gh[:, 2 * H:])
    out_ref[...] = (1.0 - z) * n + z * emb_rel


_FINAL_R = 1000


def _final_body(agg2_ref, cnt_ref, h_ref, w_ref, tgw_ref, tgb_ref, out_ref):
    deg = jnp.sum(cnt_ref[...], axis=1)
    agg = (agg2_ref[0] + agg2_ref[1]) / jnp.maximum(deg, 1.0)[:, None]
    hn = lax.dot_general(agg, w_ref[...], (((1,), (0,)), ((), ())),
                         preferred_element_type=jnp.float32)
    hn = jnp.where(hn >= 0, hn, _RRELU_SLOPE * hn)
    gate = jax.nn.sigmoid(
        lax.dot_general(hn, tgw_ref[...], (((1,), (0,)), ((), ())),
                        preferred_element_type=jnp.float32) + tgb_ref[...])
    out_ref[...] = gate * hn + (1.0 - gate) * h_ref[...]


def kernel(edge_index, edge_type, dynamic_emb, emb_rel, W_ih, W_hh, b_ih, b_hh,
           rgcn_W, time_gate_weight, time_gate_bias):
    src = edge_index[0].reshape(NW, NCHUNK, CH)
    dst = edge_index[1].reshape(NW, NCHUNK, CH)
    et = edge_type.reshape(NW, NCHUNK, CH)
    agg1, relsum, cnt = _stage_a(dynamic_emb, src, dst, et)
    cnt_t = cnt.reshape(NW, CNTN).T

    rel_evolved = pl.pallas_call(
        _gru_body,
        out_shape=jax.ShapeDtypeStruct((NR2, H), jnp.float32),
    )(emb_rel, relsum[:, :NR2, :], cnt_t[NUM_ENTS:, :],
      W_ih, W_hh, b_ih.reshape(1, 3 * H), b_hh.reshape(1, 3 * H))

    agg2 = _stage_c(rel_evolved, dst, et, agg1)

    out = pl.pallas_call(
        _final_body,
        grid=(NUM_ENTS // _FINAL_R,),
        in_specs=[
            pl.BlockSpec((NC, _FINAL_R, H), lambda i: (0, i, 0)),
            pl.BlockSpec((_FINAL_R, NW), lambda i: (i, 0)),
            pl.BlockSpec((_FINAL_R, H), lambda i: (i, 0)),
            pl.BlockSpec((H, H), lambda i: (0, 0)),
            pl.BlockSpec((H, H), lambda i: (0, 0)),
            pl.BlockSpec((1, H), lambda i: (0, 0)),
        ],
        out_specs=pl.BlockSpec((_FINAL_R, H), lambda i: (i, 0)),
        out_shape=jax.ShapeDtypeStruct((NUM_ENTS, H), jnp.float32),
    )(agg2, cnt_t[:NUM_ENTS, :], dynamic_emb, rgcn_W, time_gate_weight,
      time_gate_bias.reshape(1, H))
    return out
